# baseline (speedup 1.0000x reference)
import functools

import jax
import jax.numpy as jnp
from jax import lax
from jax.experimental import pallas as pl
from jax.experimental.pallas import tpu as pltpu
from jax.experimental.pallas import tpu_sc as plsc

VOCAB = 100000
DIM = 64
BATCH = 4096
SEQ = 50

NC = 2
NS = 16
NW = NC * NS

S_PER_W = BATCH // NW
CH_S = 2
CH_IDX = CH_S * SEQ
N_CH = S_PER_W // CH_S
NLANE = 16
D_REG = DIM // NLANE


NBUF = 4


def _accumulate(rows_v, out_v, b, c):
    for s2 in range(CH_S):
        for d in range(D_REG):
            col = pl.ds(d * NLANE, NLANE)
            vals = [rows_v[b, s2 * SEQ + j, col] for j in range(SEQ)]
            while len(vals) > 1:
                nxt = [vals[i] + vals[i + 1] for i in range(0, len(vals) - 1, 2)]
                if len(vals) % 2:
                    nxt.append(vals[-1])
                vals = nxt
            out_v[c * CH_S + s2, col] = vals[0]


def _body(idx_hbm, table_hbm, out_hbm, idx_v, rows_v, out_v, *sems):
    wid = lax.axis_index("c") * NS + lax.axis_index("s")
    pltpu.sync_copy(idx_hbm.at[pl.ds(wid * N_CH, N_CH)], idx_v)

    def issue(c, b):
        pltpu.async_copy(table_hbm.at[idx_v.at[c]], rows_v.at[b], sems[b])

    def drain(b):
        pltpu.make_async_copy(table_hbm.at[idx_v.at[0]], rows_v.at[b], sems[b]).wait()

    for b in range(NBUF):
        issue(b, b)

    def group(g, _):
        c0 = g * NBUF
        for b in range(NBUF):
            drain(b)
            _accumulate(rows_v, out_v, b, c0 + b)
            issue(c0 + b + NBUF, b)
        return ()

    lax.fori_loop(0, N_CH // NBUF - 1, group, (), unroll=False)
    for b in range(NBUF):
        drain(b)
        _accumulate(rows_v, out_v, b, N_CH - NBUF + b)

    pltpu.sync_copy(out_v, out_hbm.at[pl.ds(wid * S_PER_W, S_PER_W)])


@jax.jit
def _run(idx2d, table):
    mesh = plsc.VectorSubcoreMesh(
        core_axis_name="c", subcore_axis_name="s", num_cores=NC, num_subcores=NS
    )
    f = pl.kernel(
        _body,
        out_type=jax.ShapeDtypeStruct((BATCH, DIM), jnp.float32),
        mesh=mesh,
        scratch_types=[
            pltpu.VMEM((N_CH, CH_IDX), jnp.int32),
            pltpu.VMEM((NBUF, CH_IDX, DIM), jnp.float32),
            pltpu.VMEM((S_PER_W, DIM), jnp.float32),
        ] + [pltpu.SemaphoreType.DMA] * NBUF,
        compiler_params=pltpu.CompilerParams(use_tc_tiling_on_sc=False),
    )
    return f(idx2d, table)


def kernel(indices, table):
    idx2d = indices.astype(jnp.int32).reshape(BATCH * SEQ // CH_IDX, CH_IDX)
    return _run(idx2d, table)


# speedup vs baseline: 1.1186x; 1.1186x over previous
import functools

import jax
import jax.numpy as jnp
from jax import lax
from jax.experimental import pallas as pl
from jax.experimental.pallas import tpu as pltpu
from jax.experimental.pallas import tpu_sc as plsc

VOCAB = 100000
DIM = 64
BATCH = 4096
SEQ = 50

NC = 2
NS = 16
NW = NC * NS

S_PER_W = BATCH // NW
CH_S = 2
CH_IDX = CH_S * SEQ
N_CH = S_PER_W // CH_S
NLANE = 16
D_REG = DIM // NLANE


NBUF = 2


def _accumulate(rows_v, out_v, b, c):
    for s2 in range(CH_S):
        for d in range(D_REG):
            col = pl.ds(d * NLANE, NLANE)
            vals = [rows_v[b, s2 * SEQ + j, col] for j in range(SEQ)]
            while len(vals) > 1:
                nxt = [vals[i] + vals[i + 1] for i in range(0, len(vals) - 1, 2)]
                if len(vals) % 2:
                    nxt.append(vals[-1])
                vals = nxt
            out_v[c * CH_S + s2, col] = vals[0]


def _body(idx_hbm, table_hbm, out_hbm, idx_v, rows_v, out_v, *sems):
    wid = lax.axis_index("c") * NS + lax.axis_index("s")
    pltpu.sync_copy(idx_hbm.at[pl.ds(wid * N_CH, N_CH)], idx_v)

    def issue(c, b):
        pltpu.async_copy(table_hbm.at[idx_v.at[c]], rows_v.at[b], sems[b])

    def drain(b):
        pltpu.make_async_copy(table_hbm.at[idx_v.at[0]], rows_v.at[b], sems[b]).wait()

    for b in range(NBUF):
        issue(b, b)

    def group(g, _):
        c0 = g * NBUF
        for b in range(NBUF):
            drain(b)
            _accumulate(rows_v, out_v, b, c0 + b)
            issue(c0 + b + NBUF, b)
        return ()

    lax.fori_loop(0, N_CH // NBUF - 1, group, (), unroll=False)
    for b in range(NBUF):
        drain(b)
        _accumulate(rows_v, out_v, b, N_CH - NBUF + b)

    pltpu.sync_copy(out_v, out_hbm.at[pl.ds(wid * S_PER_W, S_PER_W)])


@jax.jit
def _run(idx2d, table):
    mesh = plsc.VectorSubcoreMesh(
        core_axis_name="c", subcore_axis_name="s", num_cores=NC, num_subcores=NS
    )
    f = pl.kernel(
        _body,
        out_type=jax.ShapeDtypeStruct((BATCH, DIM), jnp.float32),
        mesh=mesh,
        scratch_types=[
            pltpu.VMEM((N_CH, CH_IDX), jnp.int32),
            pltpu.VMEM((NBUF, CH_IDX, DIM), jnp.float32),
            pltpu.VMEM((S_PER_W, DIM), jnp.float32),
        ] + [pltpu.SemaphoreType.DMA] * NBUF,
        compiler_params=pltpu.CompilerParams(use_tc_tiling_on_sc=False),
    )
    return f(idx2d, table)


def kernel(indices, table):
    idx2d = indices.astype(jnp.int32).reshape(BATCH * SEQ // CH_IDX, CH_IDX)
    return _run(idx2d, table)


# speedup vs baseline: 1.2270x; 1.0968x over previous
import functools

import jax
import jax.numpy as jnp
from jax import lax
from jax.experimental import pallas as pl
from jax.experimental.pallas import tpu as pltpu
from jax.experimental.pallas import tpu_sc as plsc

VOCAB = 100000
DIM = 64
BATCH = 4096
SEQ = 50

NC = 2
NS = 16
NW = NC * NS

S_PER_W = BATCH // NW
CH_S = 4
CH_IDX = CH_S * SEQ
N_CH = S_PER_W // CH_S
NLANE = 16
D_REG = DIM // NLANE


NBUF = 2


def _accumulate(rows_v, out_v, b, c):
    for s2 in range(CH_S):
        for d in range(D_REG):
            col = pl.ds(d * NLANE, NLANE)
            vals = [rows_v[b, s2 * SEQ + j, col] for j in range(SEQ)]
            while len(vals) > 1:
                nxt = [vals[i] + vals[i + 1] for i in range(0, len(vals) - 1, 2)]
                if len(vals) % 2:
                    nxt.append(vals[-1])
                vals = nxt
            out_v[c * CH_S + s2, col] = vals[0]


def _body(idx_hbm, table_hbm, out_hbm, idx_v, rows_v, out_v, sem):
    wid = lax.axis_index("c") * NS + lax.axis_index("s")
    pltpu.sync_copy(idx_hbm.at[pl.ds(wid * N_CH * CH_IDX, N_CH * CH_IDX)], idx_v)

    def issue(c, b):
        pltpu.async_copy(table_hbm.at[idx_v.at[pl.ds(c * CH_IDX, CH_IDX)]], rows_v.at[b], sem.at[b])

    def drain(b):
        pltpu.make_async_copy(table_hbm.at[idx_v.at[pl.ds(0, CH_IDX)]], rows_v.at[b], sem.at[b]).wait()

    issue(0, 0)

    def chunk(c, _):
        b = lax.rem(c, NBUF)
        nb = lax.rem(c + 1, NBUF)

        @pl.when(c + 1 < N_CH)
        def _():
            issue(c + 1, nb)

        drain(b)
        _accumulate(rows_v, out_v, b, c)
        return ()

    lax.fori_loop(0, N_CH, chunk, (), unroll=False)

    pltpu.sync_copy(out_v, out_hbm.at[pl.ds(wid * S_PER_W, S_PER_W)])


@jax.jit
def _run(idx2d, table):
    mesh = plsc.VectorSubcoreMesh(
        core_axis_name="c", subcore_axis_name="s", num_cores=NC, num_subcores=NS
    )
    f = pl.kernel(
        _body,
        out_type=jax.ShapeDtypeStruct((BATCH, DIM), jnp.float32),
        mesh=mesh,
        scratch_types=[
            pltpu.VMEM((N_CH * CH_IDX,), jnp.int32),
            pltpu.VMEM((NBUF, CH_IDX, DIM), jnp.float32),
            pltpu.VMEM((S_PER_W, DIM), jnp.float32),
            pltpu.SemaphoreType.DMA((NBUF,)),
        ],
        compiler_params=pltpu.CompilerParams(use_tc_tiling_on_sc=False),
    )
    return f(idx2d, table)


def kernel(indices, table):
    idx_flat = indices.astype(jnp.int32).reshape(BATCH * SEQ)
    return _run(idx_flat, table)


# speedup vs baseline: 1.3163x; 1.0728x over previous
import functools

import jax
import jax.numpy as jnp
from jax import lax
from jax.experimental import pallas as pl
from jax.experimental.pallas import tpu as pltpu
from jax.experimental.pallas import tpu_sc as plsc

VOCAB = 100000
DIM = 64
BATCH = 4096
SEQ = 50

NC = 2
NS = 16
NW = NC * NS

S_PER_W = BATCH // NW
CH_S = 4
CH_IDX = CH_S * SEQ
N_CH = S_PER_W // CH_S
NLANE = 16
D_REG = DIM // NLANE


NBUF = 4


def _accumulate(rows_v, out_v, b, c):
    for s2 in range(CH_S):
        for d in range(D_REG):
            col = pl.ds(d * NLANE, NLANE)
            vals = [rows_v[b, s2 * SEQ + j, col] for j in range(SEQ)]
            while len(vals) > 1:
                nxt = [vals[i] + vals[i + 1] for i in range(0, len(vals) - 1, 2)]
                if len(vals) % 2:
                    nxt.append(vals[-1])
                vals = nxt
            out_v[c * CH_S + s2, col] = vals[0]


def _body(idx_hbm, table_hbm, out_hbm, idx_v, rows_v, out_v, sem):
    wid = lax.axis_index("c") * NS + lax.axis_index("s")
    pltpu.sync_copy(idx_hbm.at[pl.ds(wid * N_CH * CH_IDX, N_CH * CH_IDX)], idx_v)

    def issue(c, b):
        pltpu.async_copy(table_hbm.at[idx_v.at[pl.ds(c * CH_IDX, CH_IDX)]], rows_v.at[b], sem.at[b])

    def drain(b):
        pltpu.make_async_copy(table_hbm.at[idx_v.at[pl.ds(0, CH_IDX)]], rows_v.at[b], sem.at[b]).wait()

    for b0 in range(NBUF - 1):
        issue(b0, b0)

    def chunk(c, _):
        nb = lax.rem(c + NBUF - 1, NBUF)

        @pl.when(c + NBUF - 1 < N_CH)
        def _():
            issue(c + NBUF - 1, nb)

        b = lax.rem(c, NBUF)
        drain(b)
        _accumulate(rows_v, out_v, b, c)
        return ()

    lax.fori_loop(0, N_CH, chunk, (), unroll=False)

    pltpu.sync_copy(out_v, out_hbm.at[pl.ds(wid * S_PER_W, S_PER_W)])


@jax.jit
def _run(idx2d, table):
    mesh = plsc.VectorSubcoreMesh(
        core_axis_name="c", subcore_axis_name="s", num_cores=NC, num_subcores=NS
    )
    f = pl.kernel(
        _body,
        out_type=jax.ShapeDtypeStruct((BATCH, DIM), jnp.float32),
        mesh=mesh,
        scratch_types=[
            pltpu.VMEM((N_CH * CH_IDX,), jnp.int32),
            pltpu.VMEM((NBUF, CH_IDX, DIM), jnp.float32),
            pltpu.VMEM((S_PER_W, DIM), jnp.float32),
            pltpu.SemaphoreType.DMA((NBUF,)),
        ],
        compiler_params=pltpu.CompilerParams(use_tc_tiling_on_sc=False),
    )
    return f(idx2d, table)


def kernel(indices, table):
    idx_flat = indices.astype(jnp.int32).reshape(BATCH * SEQ)
    return _run(idx_flat, table)
